# Initial kernel scaffold; baseline (speedup 1.0000x reference)
import jax
import jax.numpy as jnp
from jax.experimental import pallas as pl


def kernel(input_ids, token_type_ids, word_table, pos_table, type_table, gamma, beta):
    raise NotImplementedError("write your pallas kernel here")


# speedup vs baseline: 2.7878x; 2.7878x over previous
import functools

import jax
import jax.numpy as jnp
from jax import lax
from jax.experimental import pallas as pl
from jax.experimental.pallas import tpu as pltpu
from jax.experimental.pallas import tpu_sc as plsc

_EPS = 1e-12


def _gather_sc(word_table, flat_ids):
    n = flat_ids.shape[0]
    h = word_table.shape[1]
    info = plsc.get_sparse_core_info()
    nc, ns = info.num_cores, info.num_subcores
    nw = nc * ns
    per_w = n // nw
    chunk = 64
    nchunks = per_w // chunk
    mesh = plsc.VectorSubcoreMesh(core_axis_name="c", subcore_axis_name="s")

    @functools.partial(
        pl.kernel,
        mesh=mesh,
        out_type=jax.ShapeDtypeStruct((n, h), jnp.float32),
        scratch_types=[
            pltpu.VMEM((per_w,), jnp.int32),
            pltpu.VMEM((chunk, h), jnp.float32),
            pltpu.SemaphoreType.DMA,
        ],
    )
    def k(ids_hbm, table_hbm, out_hbm, idx_v, rows_v, sem):
        wid = lax.axis_index("s") * nc + lax.axis_index("c")
        base = wid * per_w
        pltpu.sync_copy(ids_hbm.at[pl.ds(base, per_w)], idx_v)

        def body(c, carry):
            off = c * chunk
            pltpu.async_copy(
                table_hbm.at[idx_v.at[pl.ds(off, chunk)]], rows_v, sem
            ).wait()
            pltpu.sync_copy(rows_v, out_hbm.at[pl.ds(base + off, chunk)])
            return carry

        lax.fori_loop(0, nchunks, body, 0)

    return k(flat_ids, word_table)


def _ln_tc(gathered, tt_col, pos_table, type_table, gamma2, beta2):
    s, h = pos_table.shape
    n = gathered.shape[0]
    b = n // s

    def body(g_ref, t_ref, pos_ref, ty_ref, gm_ref, bt_ref, o_ref):
        x = g_ref[...]
        t = t_ref[...]
        ty0 = ty_ref[0:1, :]
        tyd = ty_ref[1:2, :] - ty0
        x = x + pos_ref[...] + t * tyd + ty0
        mu = jnp.mean(x, axis=1, keepdims=True)
        xc = x - mu
        var = jnp.mean(xc * xc, axis=1, keepdims=True)
        o_ref[...] = xc * lax.rsqrt(var + _EPS) * gm_ref[...] + bt_ref[...]

    return pl.pallas_call(
        body,
        grid=(b,),
        in_specs=[
            pl.BlockSpec((s, h), lambda i: (i, 0)),
            pl.BlockSpec((s, 1), lambda i: (i, 0)),
            pl.BlockSpec((s, h), lambda i: (0, 0)),
            pl.BlockSpec((2, h), lambda i: (0, 0)),
            pl.BlockSpec((1, h), lambda i: (0, 0)),
            pl.BlockSpec((1, h), lambda i: (0, 0)),
        ],
        out_specs=pl.BlockSpec((s, h), lambda i: (i, 0)),
        out_shape=jax.ShapeDtypeStruct((n, h), jnp.float32),
    )(gathered, tt_col, pos_table, type_table, gamma2, beta2)


def kernel(input_ids, token_type_ids, word_table, pos_table, type_table, gamma, beta):
    b, s = input_ids.shape
    h = word_table.shape[1]
    flat_ids = input_ids.reshape(-1).astype(jnp.int32)
    gathered = _gather_sc(word_table, flat_ids)
    tt_col = token_type_ids.reshape(-1, 1).astype(jnp.float32)
    out = _ln_tc(
        gathered,
        tt_col,
        pos_table,
        type_table,
        gamma.reshape(1, h),
        beta.reshape(1, h),
    )
    return out.reshape(b, s, h)


# speedup vs baseline: 2.9056x; 1.0422x over previous
import functools

import jax
import jax.numpy as jnp
from jax import lax
from jax.experimental import pallas as pl
from jax.experimental.pallas import tpu as pltpu
from jax.experimental.pallas import tpu_sc as plsc

_EPS = 1e-12


def _gather_sc(word_table, flat_ids):
    n = flat_ids.shape[0]
    h = word_table.shape[1]
    info = plsc.get_sparse_core_info()
    nc, ns = info.num_cores, info.num_subcores
    nw = nc * ns
    per_w = n // nw
    chunk = 64
    nchunks = per_w // chunk
    nbuf = 2
    mesh = plsc.VectorSubcoreMesh(core_axis_name="c", subcore_axis_name="s")

    @functools.partial(
        pl.kernel,
        mesh=mesh,
        out_type=jax.ShapeDtypeStruct((n, h), jnp.float32),
        scratch_types=[
            pltpu.VMEM((per_w,), jnp.int32),
            pltpu.VMEM((nbuf, chunk, h), jnp.float32),
            pltpu.SemaphoreType.DMA,
            pltpu.SemaphoreType.DMA,
            pltpu.SemaphoreType.DMA,
            pltpu.SemaphoreType.DMA,
        ],
    )
    def k(ids_hbm, table_hbm, out_hbm, idx_v, rows_v, g0, g1, s0, s1):
        gsem = (g0, g1)
        ssem = (s0, s1)
        wid = lax.axis_index("s") * nc + lax.axis_index("c")
        base = wid * per_w
        pltpu.sync_copy(ids_hbm.at[pl.ds(base, per_w)], idx_v)

        def start_gather(g):
            b = g % nbuf
            return pltpu.async_copy(
                table_hbm.at[idx_v.at[pl.ds(g * chunk, chunk)]],
                rows_v.at[b],
                gsem[b],
            )

        def start_store(g):
            b = g % nbuf
            return pltpu.async_copy(
                rows_v.at[b], out_hbm.at[pl.ds(base + g * chunk, chunk)], ssem[b]
            )

        gathers = [start_gather(0)]
        stores = [None] * nchunks
        for g in range(nchunks):
            gathers[g].wait()
            stores[g] = start_store(g)
            if g + 1 < nchunks:
                if g - 1 >= 0:
                    stores[g - 1].wait()
                gathers.append(start_gather(g + 1))
        if nchunks >= 2:
            stores[nchunks - 2].wait()
        stores[nchunks - 1].wait()

    return k(flat_ids, word_table)


def _ln_tc(gathered, tt_col, pos_table, type_table, gamma2, beta2):
    s, h = pos_table.shape
    n = gathered.shape[0]
    b = n // s

    def body(g_ref, t_ref, pos_ref, ty_ref, gm_ref, bt_ref, o_ref):
        x = g_ref[...]
        t = t_ref[...]
        ty0 = ty_ref[0:1, :]
        tyd = ty_ref[1:2, :] - ty0
        x = x + pos_ref[...] + t * tyd + ty0
        mu = jnp.mean(x, axis=1, keepdims=True)
        xc = x - mu
        var = jnp.mean(xc * xc, axis=1, keepdims=True)
        o_ref[...] = xc * lax.rsqrt(var + _EPS) * gm_ref[...] + bt_ref[...]

    return pl.pallas_call(
        body,
        grid=(b,),
        in_specs=[
            pl.BlockSpec((s, h), lambda i: (i, 0)),
            pl.BlockSpec((s, 1), lambda i: (i, 0)),
            pl.BlockSpec((s, h), lambda i: (0, 0)),
            pl.BlockSpec((2, h), lambda i: (0, 0)),
            pl.BlockSpec((1, h), lambda i: (0, 0)),
            pl.BlockSpec((1, h), lambda i: (0, 0)),
        ],
        out_specs=pl.BlockSpec((s, h), lambda i: (i, 0)),
        out_shape=jax.ShapeDtypeStruct((n, h), jnp.float32),
    )(gathered, tt_col, pos_table, type_table, gamma2, beta2)


def kernel(input_ids, token_type_ids, word_table, pos_table, type_table, gamma, beta):
    b, s = input_ids.shape
    h = word_table.shape[1]
    flat_ids = input_ids.reshape(-1).astype(jnp.int32)
    gathered = _gather_sc(word_table, flat_ids)
    tt_col = token_type_ids.reshape(-1, 1).astype(jnp.float32)
    out = _ln_tc(
        gathered,
        tt_col,
        pos_table,
        type_table,
        gamma.reshape(1, h),
        beta.reshape(1, h),
    )
    return out.reshape(b, s, h)
